# baseline (speedup 1.0000x reference)
import jax
import jax.numpy as jnp
from jax.experimental import pallas as pl

_B, _N, _NFEAT, _NHID = 8, 512, 256, 64
_G = 2


def _gcn_body(x_ref, adj_ref, w1_ref, b1_ref, w2_ref, b2_ref, w3_ref,
              b3_ref, w4_ref, b4_ref, wf_ref, bf_ref, out_ref):
    f32, bf16 = jnp.float32, jnp.bfloat16
    for i in range(_G):
        a = adj_ref[i]
        h = x_ref[i]
        for w_ref, b_ref in ((w1_ref, b1_ref), (w2_ref, b2_ref), (w3_ref, b3_ref)):
            s = jnp.dot(h, w_ref[...], preferred_element_type=f32)
            s_hi = s.astype(bf16)
            s_lo = (s - s_hi.astype(f32)).astype(bf16)
            s2 = jnp.concatenate([s_hi, s_lo], axis=1)
            g = jnp.dot(a, s2, preferred_element_type=f32)
            h = jnp.maximum(g[:, :_NHID] + g[:, _NHID:]
                            + b_ref[...], 0.0)
        v = jnp.dot(a[_N - 1:_N, :].astype(f32), h,
                    preferred_element_type=f32)
        h4 = jnp.maximum(jnp.dot(v, w4_ref[...], preferred_element_type=f32)
                         + b4_ref[...], 0.0)
        out_ref[i] = jnp.dot(h4, wf_ref[...], preferred_element_type=f32) \
            + bf_ref[...]


def kernel(x, adj, W1, b1, W2, b2, W3, b3, W4, b4, Wf, bf):
    wspec = lambda r, c: pl.BlockSpec((r, c), lambda b: (0, 0))
    out = pl.pallas_call(
        _gcn_body,
        grid=(_B // _G,),
        in_specs=[
            pl.BlockSpec((_G, _N, _NFEAT), lambda b: (b, 0, 0)),
            pl.BlockSpec((_G, _N, _N), lambda b: (b, 0, 0)),
            wspec(_NFEAT, _NHID), wspec(1, _NHID),
            wspec(_NHID, _NHID), wspec(1, _NHID),
            wspec(_NHID, _NHID), wspec(1, _NHID),
            wspec(_NHID, _NHID), wspec(1, _NHID),
            wspec(_NHID, 1), wspec(1, 1),
        ],
        out_specs=pl.BlockSpec((_G, 1, 1), lambda b: (b, 0, 0)),
        out_shape=jax.ShapeDtypeStruct((_B, 1, 1), jnp.float32),
    )(x, adj.astype(jnp.bfloat16),
      W1, b1.reshape(1, _NHID), W2, b2.reshape(1, _NHID),
      W3, b3.reshape(1, _NHID), W4, b4.reshape(1, _NHID),
      Wf, bf.reshape(1, 1))
    return out.reshape(_B, 1)


# speedup vs baseline: 1.1391x; 1.1391x over previous
import jax
import jax.numpy as jnp
from jax.experimental import pallas as pl

_B, _N, _NFEAT, _NHID = 8, 512, 256, 64


_G = 1


def _gcn_body(x_ref, adj_ref, w1_ref, b1_ref, w2_ref, b2_ref, w3_ref,
              b3_ref, w4_ref, b4_ref, wf_ref, bf_ref, out_ref):
    f32 = jnp.float32
    for i in range(_G):
        a = adj_ref[i]
        h = x_ref[i]
        for w_ref, b_ref in ((w1_ref, b1_ref), (w2_ref, b2_ref), (w3_ref, b3_ref)):
            s = jnp.dot(h, w_ref[...], preferred_element_type=f32)
            h = jnp.maximum(jnp.dot(a, s.astype(jnp.bfloat16),
                                    preferred_element_type=f32)
                            + b_ref[...], 0.0)
        v = jnp.dot(a[_N - 1:_N, :].astype(f32), h,
                    preferred_element_type=f32)
        h4 = jnp.maximum(jnp.dot(v, w4_ref[...], preferred_element_type=f32)
                         + b4_ref[...], 0.0)
        out_ref[i] = jnp.dot(h4, wf_ref[...], preferred_element_type=f32) \
            + bf_ref[...]


def kernel(x, adj, W1, b1, W2, b2, W3, b3, W4, b4, Wf, bf):
    wspec = lambda r, c: pl.BlockSpec((r, c), lambda b: (0, 0))
    out = pl.pallas_call(
        _gcn_body,
        grid=(_B // _G,),
        in_specs=[
            pl.BlockSpec((_G, _N, _NFEAT), lambda b: (b, 0, 0)),
            pl.BlockSpec((_G, _N, _N), lambda b: (b, 0, 0)),
            wspec(_NFEAT, _NHID), wspec(1, _NHID),
            wspec(_NHID, _NHID), wspec(1, _NHID),
            wspec(_NHID, _NHID), wspec(1, _NHID),
            wspec(_NHID, _NHID), wspec(1, _NHID),
            wspec(_NHID, 1), wspec(1, 1),
        ],
        out_specs=pl.BlockSpec((_G, 1, 1), lambda b: (b, 0, 0)),
        out_shape=jax.ShapeDtypeStruct((_B, 1, 1), jnp.float32),
    )(x, adj.astype(jnp.bfloat16),
      W1, b1.reshape(1, _NHID), W2, b2.reshape(1, _NHID),
      W3, b3.reshape(1, _NHID), W4, b4.reshape(1, _NHID),
      Wf, bf.reshape(1, 1))
    return out.reshape(_B, 1)


# speedup vs baseline: 1.3831x; 1.2142x over previous
import jax
import jax.numpy as jnp
from jax.experimental import pallas as pl
from jax.experimental.pallas import tpu as pltpu

_B, _N, _NFEAT, _NHID = 8, 512, 256, 64


def _gcn_body(x_hbm, adj_hbm, w1_ref, b1_ref, w2_ref, b2_ref, w3_ref,
              b3_ref, w4_ref, b4_ref, wf_ref, bf_ref, out_ref,
              x_buf, a_buf, sems):
    f32 = jnp.float32
    i = pl.program_id(0)
    slot = jax.lax.rem(i, 2)
    nslot = jax.lax.rem(i + 1, 2)

    @pl.when(i == 0)
    def _():
        pltpu.make_async_copy(x_hbm.at[0], x_buf.at[0], sems.at[0, 0]).start()
        pltpu.make_async_copy(adj_hbm.at[0], a_buf.at[0], sems.at[0, 1]).start()

    @pl.when(i + 1 < _B)
    def _():
        pltpu.make_async_copy(x_hbm.at[i + 1], x_buf.at[nslot],
                              sems.at[nslot, 0]).start()
        pltpu.make_async_copy(adj_hbm.at[i + 1], a_buf.at[nslot],
                              sems.at[nslot, 1]).start()

    pltpu.make_async_copy(x_hbm.at[i], x_buf.at[slot], sems.at[slot, 0]).wait()
    pltpu.make_async_copy(adj_hbm.at[i], a_buf.at[slot], sems.at[slot, 1]).wait()

    a = a_buf[slot]
    h = x_buf[slot]
    for w_ref, b_ref in ((w1_ref, b1_ref), (w2_ref, b2_ref), (w3_ref, b3_ref)):
        s = jnp.dot(h, w_ref[...], preferred_element_type=f32)
        h = jnp.maximum(jnp.dot(a, s, preferred_element_type=f32)
                        + b_ref[...], 0.0)
    v = jnp.dot(a[_N - 1:_N, :], h, preferred_element_type=f32)
    h4 = jnp.maximum(jnp.dot(v, w4_ref[...], preferred_element_type=f32)
                     + b4_ref[...], 0.0)
    out_ref[0] = jnp.dot(h4, wf_ref[...], preferred_element_type=f32) \
        + bf_ref[...]


def kernel(x, adj, W1, b1, W2, b2, W3, b3, W4, b4, Wf, bf):
    wspec = lambda r, c: pl.BlockSpec((r, c), lambda b: (0, 0))
    out = pl.pallas_call(
        _gcn_body,
        grid=(_B,),
        in_specs=[
            pl.BlockSpec(memory_space=pl.ANY),
            pl.BlockSpec(memory_space=pl.ANY),
            wspec(_NFEAT, _NHID), wspec(1, _NHID),
            wspec(_NHID, _NHID), wspec(1, _NHID),
            wspec(_NHID, _NHID), wspec(1, _NHID),
            wspec(_NHID, _NHID), wspec(1, _NHID),
            wspec(_NHID, 1), wspec(1, 1),
        ],
        out_specs=pl.BlockSpec((1, 1, 1), lambda b: (b, 0, 0)),
        out_shape=jax.ShapeDtypeStruct((_B, 1, 1), jnp.float32),
        scratch_shapes=[
            pltpu.VMEM((2, _N, _NFEAT), jnp.float32),
            pltpu.VMEM((2, _N, _N), jnp.float32),
            pltpu.SemaphoreType.DMA((2, 2)),
        ],
    )(x, adj,
      W1, b1.reshape(1, _NHID), W2, b2.reshape(1, _NHID),
      W3, b3.reshape(1, _NHID), W4, b4.reshape(1, _NHID),
      Wf, bf.reshape(1, 1))
    return out.reshape(_B, 1)


# speedup vs baseline: 1.6254x; 1.1752x over previous
import jax
import jax.numpy as jnp
from jax import lax
from jax.experimental import pallas as pl
from jax.experimental.pallas import tpu as pltpu

_B, _N, _NFEAT, _NHID = 8, 512, 256, 64


def _gcn_body(x_hbm, adj_hbm, w1t_ref, b1_ref, w2_ref, b2_ref, w3_ref,
              b3_ref, w4_ref, b4_ref, wf_ref, bf_ref, out_ref,
              x_buf, a_buf, sems):
    f32 = jnp.float32
    i = pl.program_id(0)
    slot = lax.rem(i, 2)
    nslot = lax.rem(i + 1, 2)

    @pl.when(i == 0)
    def _():
        pltpu.make_async_copy(x_hbm.at[0], x_buf.at[0], sems.at[0, 0]).start()
        pltpu.make_async_copy(adj_hbm.at[0], a_buf.at[0], sems.at[0, 1]).start()

    @pl.when(i + 1 < _B)
    def _():
        pltpu.make_async_copy(x_hbm.at[i + 1], x_buf.at[nslot],
                              sems.at[nslot, 0]).start()
        pltpu.make_async_copy(adj_hbm.at[i + 1], a_buf.at[nslot],
                              sems.at[nslot, 1]).start()

    pltpu.make_async_copy(x_hbm.at[i], x_buf.at[slot], sems.at[slot, 0]).wait()
    pltpu.make_async_copy(adj_hbm.at[i], a_buf.at[slot], sems.at[slot, 1]).wait()

    a = a_buf[slot]
    h = x_buf[slot]
    s = lax.dot_general(h, w1t_ref[...], (((1,), (1,)), ((), ())),
                        preferred_element_type=f32)
    h = jnp.maximum(jnp.dot(a, s, preferred_element_type=f32)
                    + b1_ref[...], 0.0)
    for w_ref, b_ref in ((w2_ref, b2_ref), (w3_ref, b3_ref)):
        s = jnp.dot(h, w_ref[...], preferred_element_type=f32)
        h = jnp.maximum(jnp.dot(a, s, preferred_element_type=f32)
                        + b_ref[...], 0.0)
    v = jnp.dot(a[_N - 1:_N, :], h, preferred_element_type=f32)
    h4 = jnp.maximum(jnp.dot(v, w4_ref[...], preferred_element_type=f32)
                     + b4_ref[...], 0.0)
    out_ref[pl.ds(i, 1), :] = jnp.sum(h4 * wf_ref[...], axis=1,
                                      keepdims=True) + bf_ref[...]


def kernel(x, adj, W1, b1, W2, b2, W3, b3, W4, b4, Wf, bf):
    wspec = lambda r, c: pl.BlockSpec((r, c), lambda b: (0, 0))
    out = pl.pallas_call(
        _gcn_body,
        grid=(_B,),
        in_specs=[
            pl.BlockSpec(memory_space=pl.ANY),
            pl.BlockSpec(memory_space=pl.ANY),
            wspec(_NHID, _NFEAT), wspec(1, _NHID),
            wspec(_NHID, _NHID), wspec(1, _NHID),
            wspec(_NHID, _NHID), wspec(1, _NHID),
            wspec(_NHID, _NHID), wspec(1, _NHID),
            wspec(1, _NHID), wspec(1, 1),
        ],
        out_specs=pl.BlockSpec((_B, 1), lambda b: (0, 0)),
        out_shape=jax.ShapeDtypeStruct((_B, 1), jnp.float32),
        scratch_shapes=[
            pltpu.VMEM((2, _N, _NFEAT), jnp.float32),
            pltpu.VMEM((2, _N, _N), jnp.float32),
            pltpu.SemaphoreType.DMA((2, 2)),
        ],
    )(x, adj,
      W1.T, b1.reshape(1, _NHID), W2, b2.reshape(1, _NHID),
      W3, b3.reshape(1, _NHID), W4, b4.reshape(1, _NHID),
      Wf.reshape(1, _NHID), bf.reshape(1, 1))
    return out


# speedup vs baseline: 1.7474x; 1.0750x over previous
import jax
import jax.numpy as jnp
from jax import lax
from jax.experimental import pallas as pl
from jax.experimental.pallas import tpu as pltpu

_B, _N, _NFEAT, _NHID = 8, 512, 256, 64
_G = 2
_S = _B // _G


def _gcn_body(x_hbm, adj_hbm, w1t_ref, b1_ref, w2_ref, b2_ref, w3_ref,
              b3_ref, w4_ref, b4_ref, wf_ref, bf_ref, out_ref,
              x_buf, a_buf, sems):
    f32 = jnp.float32
    i = pl.program_id(0)
    slot = lax.rem(i, 2)
    nslot = lax.rem(i + 1, 2)

    @pl.when(i == 0)
    def _():
        pltpu.make_async_copy(x_hbm.at[pl.ds(0, _G)], x_buf.at[0],
                              sems.at[0, 0]).start()
        pltpu.make_async_copy(adj_hbm.at[pl.ds(0, _G)], a_buf.at[0],
                              sems.at[0, 1]).start()

    @pl.when(i + 1 < _S)
    def _():
        pltpu.make_async_copy(x_hbm.at[pl.ds((i + 1) * _G, _G)],
                              x_buf.at[nslot], sems.at[nslot, 0]).start()
        pltpu.make_async_copy(adj_hbm.at[pl.ds((i + 1) * _G, _G)],
                              a_buf.at[nslot], sems.at[nslot, 1]).start()

    pltpu.make_async_copy(x_hbm.at[pl.ds(i * _G, _G)], x_buf.at[slot],
                          sems.at[slot, 0]).wait()
    pltpu.make_async_copy(adj_hbm.at[pl.ds(i * _G, _G)], a_buf.at[slot],
                          sems.at[slot, 1]).wait()

    @pl.when(i == 0)
    def _():
        out_ref[...] = jnp.zeros((1, _B), f32)

    lane = lax.broadcasted_iota(jnp.int32, (1, _B), 1)
    for g in range(_G):
        a = a_buf[slot, g]
        h = x_buf[slot, g]
        s = lax.dot_general(h, w1t_ref[...], (((1,), (1,)), ((), ())),
                            preferred_element_type=f32)
        h = jnp.maximum(jnp.dot(a, s, preferred_element_type=f32)
                        + b1_ref[...], 0.0)
        for w_ref, b_ref in ((w2_ref, b2_ref), (w3_ref, b3_ref)):
            s = jnp.dot(h, w_ref[...], preferred_element_type=f32)
            h = jnp.maximum(jnp.dot(a, s, preferred_element_type=f32)
                            + b_ref[...], 0.0)
        v = jnp.dot(a[_N - 1:_N, :], h, preferred_element_type=f32)
        h4 = jnp.maximum(jnp.dot(v, w4_ref[...], preferred_element_type=f32)
                         + b4_ref[...], 0.0)
        val = jnp.sum(h4 * wf_ref[...], axis=1, keepdims=True) \
            + bf_ref[...]
        out_ref[...] += jnp.where(lane == i * _G + g, val, 0.0)


def kernel(x, adj, W1, b1, W2, b2, W3, b3, W4, b4, Wf, bf):
    wspec = lambda r, c: pl.BlockSpec((r, c), lambda b: (0, 0))
    out = pl.pallas_call(
        _gcn_body,
        grid=(_S,),
        in_specs=[
            pl.BlockSpec(memory_space=pl.ANY),
            pl.BlockSpec(memory_space=pl.ANY),
            wspec(_NHID, _NFEAT), wspec(1, _NHID),
            wspec(_NHID, _NHID), wspec(1, _NHID),
            wspec(_NHID, _NHID), wspec(1, _NHID),
            wspec(_NHID, _NHID), wspec(1, _NHID),
            wspec(1, _NHID), wspec(1, 1),
        ],
        out_specs=pl.BlockSpec((1, _B), lambda b: (0, 0)),
        out_shape=jax.ShapeDtypeStruct((1, _B), jnp.float32),
        scratch_shapes=[
            pltpu.VMEM((2, _G, _N, _NFEAT), jnp.float32),
            pltpu.VMEM((2, _G, _N, _N), jnp.float32),
            pltpu.SemaphoreType.DMA((2, 2)),
        ],
    )(x, adj,
      W1.T, b1.reshape(1, _NHID), W2, b2.reshape(1, _NHID),
      W3, b3.reshape(1, _NHID), W4, b4.reshape(1, _NHID),
      Wf.reshape(1, _NHID), bf.reshape(1, 1))
    return out.T


# speedup vs baseline: 2.5403x; 1.4538x over previous
import jax
import jax.numpy as jnp
from jax import lax
from jax.experimental import pallas as pl
from jax.experimental.pallas import tpu as pltpu

_B, _N, _NFEAT, _NHID = 8, 512, 256, 64
_G = 2
_S = _B // _G


def _gcn_body(x_hbm, adj_hbm, w1t_ref, b1_ref, w2_ref, b2_ref, w3_ref,
              b3_ref, w4_ref, b4_ref, wf_ref, bf_ref, out_ref,
              x_buf, a_buf, sems):
    f32 = jnp.float32
    i = pl.program_id(0)
    slot = lax.rem(i, 2)
    nslot = lax.rem(i + 1, 2)

    @pl.when(i == 0)
    def _():
        pltpu.make_async_copy(x_hbm.at[pl.ds(0, _G)], x_buf.at[0],
                              sems.at[0, 0]).start()
        pltpu.make_async_copy(adj_hbm.at[pl.ds(0, _G)], a_buf.at[0],
                              sems.at[0, 1]).start()

    @pl.when(i + 1 < _S)
    def _():
        pltpu.make_async_copy(x_hbm.at[pl.ds((i + 1) * _G, _G)],
                              x_buf.at[nslot], sems.at[nslot, 0]).start()
        pltpu.make_async_copy(adj_hbm.at[pl.ds((i + 1) * _G, _G)],
                              a_buf.at[nslot], sems.at[nslot, 1]).start()

    pltpu.make_async_copy(x_hbm.at[pl.ds(i * _G, _G)], x_buf.at[slot],
                          sems.at[slot, 0]).wait()
    pltpu.make_async_copy(adj_hbm.at[pl.ds(i * _G, _G)], a_buf.at[slot],
                          sems.at[slot, 1]).wait()

    @pl.when(i == 0)
    def _():
        out_ref[...] = jnp.zeros((1, _B), f32)

    lane = lax.broadcasted_iota(jnp.int32, (1, _B), 1)
    aa = [a_buf[slot, g] for g in range(_G)]
    h_all = x_buf[slot].reshape(_G * _N, _NFEAT)
    s_all = lax.dot_general(h_all, w1t_ref[...], (((1,), (1,)), ((), ())),
                            preferred_element_type=f32)
    for w_ref, b_ref in ((w2_ref, b1_ref), (w3_ref, b2_ref), (None, b3_ref)):
        g_parts = [jnp.dot(aa[g], s_all[g * _N:(g + 1) * _N],
                           preferred_element_type=f32) for g in range(_G)]
        h_all = jnp.maximum(jnp.concatenate(g_parts, axis=0)
                            + b_ref[...], 0.0)
        if w_ref is not None:
            s_all = jnp.dot(h_all, w_ref[...], preferred_element_type=f32)
    for g in range(_G):
        v = jnp.dot(aa[g][_N - 1:_N, :], h_all[g * _N:(g + 1) * _N],
                    preferred_element_type=f32)
        h4 = jnp.maximum(jnp.dot(v, w4_ref[...], preferred_element_type=f32)
                         + b4_ref[...], 0.0)
        val = jnp.sum(h4 * wf_ref[...], axis=1, keepdims=True) \
            + bf_ref[...]
        out_ref[...] += jnp.where(lane == i * _G + g, val, 0.0)


def kernel(x, adj, W1, b1, W2, b2, W3, b3, W4, b4, Wf, bf):
    wspec = lambda r, c: pl.BlockSpec((r, c), lambda b: (0, 0))
    out = pl.pallas_call(
        _gcn_body,
        grid=(_S,),
        in_specs=[
            pl.BlockSpec(memory_space=pl.ANY),
            pl.BlockSpec(memory_space=pl.ANY),
            wspec(_NHID, _NFEAT), wspec(1, _NHID),
            wspec(_NHID, _NHID), wspec(1, _NHID),
            wspec(_NHID, _NHID), wspec(1, _NHID),
            wspec(_NHID, _NHID), wspec(1, _NHID),
            wspec(1, _NHID), wspec(1, 1),
        ],
        out_specs=pl.BlockSpec((1, _B), lambda b: (0, 0)),
        out_shape=jax.ShapeDtypeStruct((1, _B), jnp.float32),
        scratch_shapes=[
            pltpu.VMEM((2, _G, _N, _NFEAT), jnp.float32),
            pltpu.VMEM((2, _G, _N, _N), jnp.float32),
            pltpu.SemaphoreType.DMA((2, 2)),
        ],
    )(x, adj,
      W1.T, b1.reshape(1, _NHID), W2, b2.reshape(1, _NHID),
      W3, b3.reshape(1, _NHID), W4, b4.reshape(1, _NHID),
      Wf.reshape(1, _NHID), bf.reshape(1, 1))
    return out.T
